# baseline (speedup 1.0000x reference)
import jax
import jax.numpy as jnp
from jax import lax
from jax.experimental import pallas as pl
from jax.experimental.pallas import tpu as pltpu
from jax.experimental.pallas import tpu_sc as plsc

N = 10000
E = 320000
IN_F = 128
OUT_F = 16
HEADS = 8
ALPHA = 0.2

NC, NS, L = 2, 16, 16
NW = NC * NS
N_PAD = 10240
TW = IN_F + HEADS
C = 80
NCHUNK = 128
EW = NCHUNK * C
E_PAD = NW * EW
ROWS_PER_TILE = N_PAD // NS
LAST = NCHUNK - 1


def _dense_body(x_ref, wcat_ref, a_ref, t_ref, ssrc_ref):
    xb = x_ref[...]
    wh = jnp.dot(xb, wcat_ref[...], preferred_element_type=jnp.float32)
    ssrc, sdst = [], []
    for h in range(HEADS):
        whh = wh[:, h * OUT_F:(h + 1) * OUT_F]
        ssrc.append(jnp.dot(whh, a_ref[h, :OUT_F, :],
                            preferred_element_type=jnp.float32))
        sdst.append(jnp.dot(whh, a_ref[h, OUT_F:, :],
                            preferred_element_type=jnp.float32))
    bn = xb.shape[0]
    wh_hm = wh.reshape(bn, HEADS, OUT_F).transpose(0, 2, 1).reshape(bn, IN_F)
    t_ref[...] = jnp.concatenate([wh_hm] + sdst, axis=1)
    ssrc_ref[...] = jnp.concatenate(ssrc, axis=1)


def _dense(x_pad, w_cat, a):
    BN = 1024
    return pl.pallas_call(
        _dense_body,
        grid=(N_PAD // BN,),
        in_specs=[
            pl.BlockSpec((BN, IN_F), lambda i: (i, 0)),
            pl.BlockSpec((IN_F, IN_F), lambda i: (0, 0)),
            pl.BlockSpec((HEADS, 2 * OUT_F, 1), lambda i: (0, 0, 0)),
        ],
        out_specs=[
            pl.BlockSpec((BN, TW), lambda i: (i, 0)),
            pl.BlockSpec((BN, HEADS), lambda i: (i, 0)),
        ],
        out_shape=[
            jax.ShapeDtypeStruct((N_PAD, TW), jnp.float32),
            jax.ShapeDtypeStruct((N_PAD, HEADS), jnp.float32),
        ],
    )(x_pad, w_cat, a)


def _sc_body(t_hbm, ssrc_hbm, src_hbm, dst_hbm, out_hbm,
             src_v, dst_v, gs_v, rows_v, acc, sem_r, sem_g, sem_sr):
    c_idx = lax.axis_index("c")
    s_idx = lax.axis_index("s")
    wid = s_idx * NC + c_idx

    iota = lax.iota(jnp.int32, L)
    row_base = iota >> 3
    col_lo = iota & 7
    z16 = jnp.zeros((L,), jnp.float32)

    pltpu.sync_copy(src_hbm.at[wid], src_v)
    pltpu.sync_copy(dst_hbm.at[wid], dst_v)

    def _zrow(k, carry):
        ridx = 2 * k + row_base
        for s in range(2):
            for r in range(2):
                for cc in range(IN_F // L):
                    rows_v[s, 2 * k + r, pl.ds(cc * L, L)] = z16
            plsc.store_scatter(rows_v.at[s], [ridx, IN_F + col_lo], z16)
        return carry
    lax.fori_loop(0, C // 2, _zrow, 0)

    base = s_idx * ROWS_PER_TILE
    for b in range(ROWS_PER_TILE // C):
        pltpu.sync_copy(rows_v.at[0], acc.at[pl.ds(base + b * C, C)])
    plsc.subcore_barrier()

    pltpu.async_copy(rows_v.at[1], acc.at[src_v.at[0]], sem_sr, add=True)
    pltpu.async_copy(t_hbm.at[dst_v.at[0]], rows_v.at[0], sem_r)
    pltpu.async_copy(ssrc_hbm.at[src_v.at[0]], gs_v, sem_g)

    def _chunk(j, carry):
        p = j & 1
        pn = 1 - p
        jn = jnp.minimum(j + 1, LAST)

        pltpu.make_async_copy(t_hbm.at[dst_v.at[j]], rows_v.at[p],
                              sem_r).wait()
        pltpu.make_async_copy(ssrc_hbm.at[src_v.at[j]], gs_v, sem_g).wait()

        def _wbody(k, carry2):
            ridx = 2 * k + row_base
            s1 = plsc.load_gather(gs_v, [ridx, col_lo])
            s2 = plsc.load_gather(rows_v.at[p], [ridx, IN_F + col_lo])
            z = s1 + s2
            w = jnp.exp(-jnp.where(z > 0, z, ALPHA * z))
            plsc.store_scatter(rows_v.at[p], [ridx, IN_F + col_lo], w)
            return carry2
        lax.fori_loop(0, C // 2, _wbody, 0, unroll=2)

        pltpu.async_copy(ssrc_hbm.at[src_v.at[jn]], gs_v, sem_g)

        pltpu.make_async_copy(rows_v.at[pn], acc.at[src_v.at[j]],
                              sem_sr).wait()
        pltpu.async_copy(t_hbm.at[dst_v.at[jn]], rows_v.at[pn], sem_r)

        def _sbody(e, carry2):
            esp = iota * 0 + e
            wp = plsc.load_gather(rows_v.at[p], [esp, IN_F + col_lo])
            for h8 in range(HEADS):
                seg = rows_v[p, e, pl.ds(h8 * L, L)]
                rows_v[p, e, pl.ds(h8 * L, L)] = seg * wp
            return carry2
        lax.fori_loop(0, C, _sbody, 0)

        pltpu.async_copy(rows_v.at[p], acc.at[src_v.at[j]], sem_sr, add=True)
        return carry

    lax.fori_loop(0, NCHUNK, _chunk, 0)

    pltpu.make_async_copy(ssrc_hbm.at[src_v.at[LAST]], gs_v, sem_g).wait()
    pltpu.make_async_copy(t_hbm.at[dst_v.at[LAST]],
                          rows_v.at[(LAST + 1) & 1], sem_r).wait()
    pltpu.make_async_copy(rows_v.at[LAST & 1], acc.at[src_v.at[LAST]],
                          sem_sr).wait()
    plsc.subcore_barrier()

    for b in range(ROWS_PER_TILE // C):
        r0 = base + b * C
        pltpu.sync_copy(acc.at[pl.ds(r0, C)], rows_v.at[0])
        pltpu.sync_copy(rows_v.at[0], out_hbm.at[c_idx, pl.ds(r0, C)])


def _sc_edge(t_tab, ssrc_tab, src_w, dst_w):
    mesh = plsc.VectorSubcoreMesh(core_axis_name="c", subcore_axis_name="s",
                                  num_cores=NC, num_subcores=NS)
    f = pl.kernel(
        _sc_body,
        out_type=jax.ShapeDtypeStruct((NC, N_PAD, TW), jnp.float32),
        mesh=mesh,
        compiler_params=pltpu.CompilerParams(needs_layout_passes=False,
                                             use_tc_tiling_on_sc=False),
        scratch_types=[
            pltpu.VMEM((NCHUNK, C), jnp.int32),
            pltpu.VMEM((NCHUNK, C), jnp.int32),
            pltpu.VMEM((C, HEADS), jnp.float32),
            pltpu.VMEM((2, C, TW), jnp.float32),
            pltpu.VMEM_SHARED((N_PAD, TW), jnp.float32),
            pltpu.SemaphoreType.DMA,
            pltpu.SemaphoreType.DMA,
            pltpu.SemaphoreType.DMA,
        ],
    )
    return f(t_tab, ssrc_tab, src_w, dst_w)


def _norm_body(acc_ref, out_ref):
    num = acc_ref[0, :, :IN_F] + acc_ref[1, :, :IN_F]
    den = acc_ref[0, :, IN_F:] + acc_ref[1, :, IN_F:]
    inv = 1.0 / (den + 1e-10)
    bn = num.shape[0]
    scaled = num.reshape(bn, OUT_F, HEADS) * inv[:, None, :]
    out_ref[...] = scaled.transpose(0, 2, 1).reshape(bn, IN_F)


def _normalize(acc):
    BN = 1024
    return pl.pallas_call(
        _norm_body,
        grid=(N_PAD // BN,),
        in_specs=[pl.BlockSpec((NC, BN, TW), lambda i: (0, i, 0))],
        out_specs=pl.BlockSpec((BN, IN_F), lambda i: (i, 0)),
        out_shape=jax.ShapeDtypeStruct((N_PAD, IN_F), jnp.float32),
    )(acc)


@jax.jit
def kernel(x, edge_index, W, a):
    x_pad = jnp.pad(x, ((0, N_PAD - N), (0, 0)))
    w_cat = jnp.transpose(W, (1, 0, 2)).reshape(IN_F, HEADS * OUT_F)

    pad_per_w = (E_PAD - E) // NW
    dummy = jnp.broadcast_to(N + jnp.arange(pad_per_w, dtype=jnp.int32),
                             (NW, pad_per_w))
    src = jnp.concatenate([edge_index[0].reshape(NW, E // NW), dummy],
                          axis=1).reshape(NW, NCHUNK, C)
    dst = jnp.concatenate([edge_index[1].reshape(NW, E // NW), dummy],
                          axis=1).reshape(NW, NCHUNK, C)

    t_tab, ssrc_tab = _dense(x_pad, w_cat, a)
    acc = _sc_edge(t_tab, ssrc_tab, src, dst)
    out = _normalize(acc)
    return out[:N]


# speedup vs baseline: 1.3002x; 1.3002x over previous
import jax
import jax.numpy as jnp
from jax import lax
from jax.experimental import pallas as pl
from jax.experimental.pallas import tpu as pltpu
from jax.experimental.pallas import tpu_sc as plsc

N = 10000
E = 320000
IN_F = 128
OUT_F = 16
HEADS = 8
ALPHA = 0.2

NC, NS, L = 2, 16, 16
NW = NC * NS
N_PAD = 10240
C = 80
NCHUNK = 128
EW = NCHUNK * C
E_PAD = NW * EW
ROWS_PER_TILE = N_PAD // NS
LAST = NCHUNK - 1


def _dense_body(x_ref, wcat_ref, a_ref, wh_ref, ssrc_ref, sdst_ref):
    xb = x_ref[...]
    wh = jnp.dot(xb, wcat_ref[...], preferred_element_type=jnp.float32)
    ssrc, sdst = [], []
    for h in range(HEADS):
        whh = wh[:, h * OUT_F:(h + 1) * OUT_F]
        ssrc.append(jnp.dot(whh, a_ref[h, :OUT_F, :],
                            preferred_element_type=jnp.float32))
        sdst.append(jnp.dot(whh, a_ref[h, OUT_F:, :],
                            preferred_element_type=jnp.float32))
    bn = xb.shape[0]
    wh_ref[...] = wh.reshape(bn, HEADS, OUT_F).transpose(0, 2, 1).reshape(
        bn, IN_F)
    ssrc_ref[...] = jnp.concatenate(ssrc, axis=1)
    sdst_ref[...] = jnp.concatenate(sdst, axis=1)


def _dense(x_pad, w_cat, a):
    BN = 1024
    return pl.pallas_call(
        _dense_body,
        grid=(N_PAD // BN,),
        in_specs=[
            pl.BlockSpec((BN, IN_F), lambda i: (i, 0)),
            pl.BlockSpec((IN_F, IN_F), lambda i: (0, 0)),
            pl.BlockSpec((HEADS, 2 * OUT_F, 1), lambda i: (0, 0, 0)),
        ],
        out_specs=[
            pl.BlockSpec((BN, IN_F), lambda i: (i, 0)),
            pl.BlockSpec((BN, HEADS), lambda i: (i, 0)),
            pl.BlockSpec((BN, HEADS), lambda i: (i, 0)),
        ],
        out_shape=[
            jax.ShapeDtypeStruct((N_PAD, IN_F), jnp.float32),
            jax.ShapeDtypeStruct((N_PAD, HEADS), jnp.float32),
            jax.ShapeDtypeStruct((N_PAD, HEADS), jnp.float32),
        ],
    )(x_pad, w_cat, a)


def _sc_body(wh_hbm, ssrc_hbm, sdst_hbm, src_hbm, dst_hbm, num_hbm, den_hbm,
             src_v, dst_v, gs_v, gd_v, w_v, rows_v,
             acc_num, acc_den, sem_r, sem_g, sem_h, sem_sr, sem_sw):
    c_idx = lax.axis_index("c")
    s_idx = lax.axis_index("s")
    wid = s_idx * NC + c_idx

    iota = lax.iota(jnp.int32, L)
    row_base = iota >> 3
    col_lo = iota & 7
    z16 = jnp.zeros((L,), jnp.float32)

    pltpu.sync_copy(src_hbm.at[wid], src_v)
    pltpu.sync_copy(dst_hbm.at[wid], dst_v)

    def _zrow(r, carry):
        for s in range(2):
            for cc in range(IN_F // L):
                rows_v[s, r, pl.ds(cc * L, L)] = z16
        return carry
    lax.fori_loop(0, C, _zrow, 0)
    def _zw(k, carry):
        for s in range(2):
            plsc.store_scatter(w_v.at[s], [2 * k + row_base, col_lo], z16)
        return carry
    lax.fori_loop(0, C // 2, _zw, 0)

    base = s_idx * ROWS_PER_TILE
    for b in range(ROWS_PER_TILE // C):
        pltpu.sync_copy(rows_v.at[0], acc_num.at[pl.ds(base + b * C, C)])
        pltpu.sync_copy(w_v.at[0], acc_den.at[pl.ds(base + b * C, C)])
    plsc.subcore_barrier()

    pltpu.async_copy(rows_v.at[1], acc_num.at[src_v.at[0]], sem_sr, add=True)
    pltpu.async_copy(w_v.at[1], acc_den.at[src_v.at[0]], sem_sw, add=True)
    pltpu.async_copy(wh_hbm.at[dst_v.at[0]], rows_v.at[0], sem_r)
    pltpu.async_copy(ssrc_hbm.at[src_v.at[0]], gs_v, sem_g)
    pltpu.async_copy(sdst_hbm.at[dst_v.at[0]], gd_v, sem_h)

    def _chunk(j, carry):
        p = j & 1
        pn = 1 - p
        jn = jnp.minimum(j + 1, LAST)

        pltpu.make_async_copy(ssrc_hbm.at[src_v.at[j]], gs_v, sem_g).wait()
        pltpu.make_async_copy(sdst_hbm.at[dst_v.at[j]], gd_v, sem_h).wait()

        def _wbody(k):
            ridx = 2 * k + row_base
            s1 = plsc.load_gather(gs_v, [ridx, col_lo])
            s2 = plsc.load_gather(gd_v, [ridx, col_lo])
            z = s1 + s2
            w = jnp.exp(-jnp.where(z > 0, z, ALPHA * z))
            plsc.store_scatter(w_v.at[p], [ridx, col_lo], w)
        plsc.parallel_loop(0, C // 2, unroll=4)(_wbody)

        pltpu.async_copy(ssrc_hbm.at[src_v.at[jn]], gs_v, sem_g)
        pltpu.async_copy(sdst_hbm.at[dst_v.at[jn]], gd_v, sem_h)

        pltpu.make_async_copy(wh_hbm.at[dst_v.at[j]], rows_v.at[p],
                              sem_r).wait()

        pltpu.make_async_copy(rows_v.at[pn], acc_num.at[src_v.at[j]],
                              sem_sr).wait()
        pltpu.make_async_copy(w_v.at[pn], acc_den.at[src_v.at[j]],
                              sem_sw).wait()
        pltpu.async_copy(wh_hbm.at[dst_v.at[jn]], rows_v.at[pn], sem_r)

        def _sbody(e):
            esp = iota * 0 + e
            wp = plsc.load_gather(w_v.at[p], [esp, col_lo])
            for h8 in range(HEADS):
                seg = rows_v[p, e, pl.ds(h8 * L, L)]
                rows_v[p, e, pl.ds(h8 * L, L)] = seg * wp
        plsc.parallel_loop(0, C, unroll=2)(_sbody)

        pltpu.async_copy(rows_v.at[p], acc_num.at[src_v.at[j]], sem_sr,
                         add=True)
        pltpu.async_copy(w_v.at[p], acc_den.at[src_v.at[j]], sem_sw,
                         add=True)
        return carry

    lax.fori_loop(0, NCHUNK, _chunk, 0)

    pltpu.make_async_copy(ssrc_hbm.at[src_v.at[LAST]], gs_v, sem_g).wait()
    pltpu.make_async_copy(sdst_hbm.at[dst_v.at[LAST]], gd_v, sem_h).wait()
    pltpu.make_async_copy(wh_hbm.at[dst_v.at[LAST]],
                          rows_v.at[(LAST + 1) & 1], sem_r).wait()
    pltpu.make_async_copy(rows_v.at[LAST & 1], acc_num.at[src_v.at[LAST]],
                          sem_sr).wait()
    pltpu.make_async_copy(w_v.at[LAST & 1], acc_den.at[src_v.at[LAST]],
                          sem_sw).wait()
    plsc.subcore_barrier()

    for b in range(ROWS_PER_TILE // C):
        r0 = base + b * C
        pltpu.sync_copy(acc_num.at[pl.ds(r0, C)], rows_v.at[0])
        pltpu.sync_copy(rows_v.at[0], num_hbm.at[c_idx, pl.ds(r0, C)])
        pltpu.sync_copy(acc_den.at[pl.ds(r0, C)], w_v.at[0])
        pltpu.sync_copy(w_v.at[0], den_hbm.at[c_idx, pl.ds(r0, C)])


def _sc_edge(wh_tab, ssrc_tab, sdst_tab, src_w, dst_w):
    mesh = plsc.VectorSubcoreMesh(core_axis_name="c", subcore_axis_name="s",
                                  num_cores=NC, num_subcores=NS)
    f = pl.kernel(
        _sc_body,
        out_type=[
            jax.ShapeDtypeStruct((NC, N_PAD, IN_F), jnp.float32),
            jax.ShapeDtypeStruct((NC, N_PAD, HEADS), jnp.float32),
        ],
        mesh=mesh,
        compiler_params=pltpu.CompilerParams(needs_layout_passes=False,
                                             use_tc_tiling_on_sc=False),
        scratch_types=[
            pltpu.VMEM((NCHUNK, C), jnp.int32),
            pltpu.VMEM((NCHUNK, C), jnp.int32),
            pltpu.VMEM((C, HEADS), jnp.float32),
            pltpu.VMEM((C, HEADS), jnp.float32),
            pltpu.VMEM((2, C, HEADS), jnp.float32),
            pltpu.VMEM((2, C, IN_F), jnp.float32),
            pltpu.VMEM_SHARED((N_PAD, IN_F), jnp.float32),
            pltpu.VMEM_SHARED((N_PAD, HEADS), jnp.float32),
            pltpu.SemaphoreType.DMA,
            pltpu.SemaphoreType.DMA,
            pltpu.SemaphoreType.DMA,
            pltpu.SemaphoreType.DMA,
            pltpu.SemaphoreType.DMA,
        ],
    )
    return f(wh_tab, ssrc_tab, sdst_tab, src_w, dst_w)


def _norm_body(num_ref, den_ref, out_ref):
    num = num_ref[0] + num_ref[1]
    den = den_ref[0] + den_ref[1]
    inv = 1.0 / (den + 1e-10)
    bn = num.shape[0]
    scaled = num.reshape(bn, OUT_F, HEADS) * inv[:, None, :]
    out_ref[...] = scaled.transpose(0, 2, 1).reshape(bn, IN_F)


def _normalize(num, den):
    BN = 1024
    return pl.pallas_call(
        _norm_body,
        grid=(N_PAD // BN,),
        in_specs=[
            pl.BlockSpec((NC, BN, IN_F), lambda i: (0, i, 0)),
            pl.BlockSpec((NC, BN, HEADS), lambda i: (0, i, 0)),
        ],
        out_specs=pl.BlockSpec((BN, IN_F), lambda i: (i, 0)),
        out_shape=jax.ShapeDtypeStruct((N_PAD, IN_F), jnp.float32),
    )(num, den)


@jax.jit
def kernel(x, edge_index, W, a):
    x_pad = jnp.pad(x, ((0, N_PAD - N), (0, 0)))
    w_cat = jnp.transpose(W, (1, 0, 2)).reshape(IN_F, HEADS * OUT_F)

    pad_per_w = (E_PAD - E) // NW
    dummy = jnp.broadcast_to(N + jnp.arange(pad_per_w, dtype=jnp.int32),
                             (NW, pad_per_w))
    src = jnp.concatenate([edge_index[0].reshape(NW, E // NW), dummy],
                          axis=1).reshape(NW, NCHUNK, C)
    dst = jnp.concatenate([edge_index[1].reshape(NW, E // NW), dummy],
                          axis=1).reshape(NW, NCHUNK, C)

    wh_tab, ssrc_tab, sdst_tab = _dense(x_pad, w_cat, a)
    num, den = _sc_edge(wh_tab, ssrc_tab, sdst_tab, src, dst)
    out = _normalize(num, den)
    return out[:N]


# speedup vs baseline: 1.8065x; 1.3894x over previous
import numpy as np

import jax
import jax.numpy as jnp
from jax import lax
from jax.experimental import pallas as pl
from jax.experimental.pallas import tpu as pltpu
from jax.experimental.pallas import tpu_sc as plsc

N = 10000
E = 320000
IN_F = 128
OUT_F = 16
HEADS = 8
ALPHA = 0.2

NC, NS, L = 2, 16, 16
NW = NC * NS
C = 80
NCHUNK = 125
EW = NCHUNK * C
ROWS_PER_TILE = N // NS
LAST = NCHUNK - 1

_PERM = np.zeros((IN_F, IN_F), np.float32)
for _k in range(OUT_F):
    for _h in range(HEADS):
        _PERM[_k * HEADS + _h, _h * OUT_F + _k] = 1.0


def _dense_body(x_ref, wcat_ref, w_ref, a_ref, wh_ref, ssrc_ref, sdst_ref):
    xb = x_ref[...]
    wh_ref[...] = jnp.dot(xb, wcat_ref[...],
                          preferred_element_type=jnp.float32)
    avecs = []
    for h in range(HEADS):
        avecs.append(jnp.dot(w_ref[h], a_ref[h, :OUT_F, :],
                             preferred_element_type=jnp.float32))
    for h in range(HEADS):
        avecs.append(jnp.dot(w_ref[h], a_ref[h, OUT_F:, :],
                             preferred_element_type=jnp.float32))
    amat = jnp.concatenate(avecs, axis=1)
    ss = jnp.dot(xb, amat, preferred_element_type=jnp.float32)
    ssrc_ref[...] = ss[:, :HEADS]
    sdst_ref[...] = ss[:, HEADS:]


def _dense(x, w_cat, W, a):
    BN = 1000
    return pl.pallas_call(
        _dense_body,
        grid=(N // BN,),
        in_specs=[
            pl.BlockSpec((BN, IN_F), lambda i: (i, 0)),
            pl.BlockSpec((IN_F, IN_F), lambda i: (0, 0)),
            pl.BlockSpec((HEADS, IN_F, OUT_F), lambda i: (0, 0, 0)),
            pl.BlockSpec((HEADS, 2 * OUT_F, 1), lambda i: (0, 0, 0)),
        ],
        out_specs=[
            pl.BlockSpec((BN, IN_F), lambda i: (i, 0)),
            pl.BlockSpec((BN, HEADS), lambda i: (i, 0)),
            pl.BlockSpec((BN, HEADS), lambda i: (i, 0)),
        ],
        out_shape=[
            jax.ShapeDtypeStruct((N, IN_F), jnp.float32),
            jax.ShapeDtypeStruct((N, HEADS), jnp.float32),
            jax.ShapeDtypeStruct((N, HEADS), jnp.float32),
        ],
    )(x, w_cat, W, a)


def _sc_body(wh_hbm, ssrc_hbm, sdst_hbm, src_hbm, dst_hbm, num_hbm, den_hbm,
             src_v, dst_v, gs_v, gd_v, w_v, rows_v,
             acc_num, acc_den, sem_r, sem_g, sem_h, sem_sr, sem_sw):
    c_idx = lax.axis_index("c")
    s_idx = lax.axis_index("s")
    wid = s_idx * NC + c_idx

    iota = lax.iota(jnp.int32, L)
    row_base = iota >> 3
    col_lo = iota & 7
    z16 = jnp.zeros((L,), jnp.float32)

    pltpu.sync_copy(src_hbm.at[wid], src_v)
    pltpu.sync_copy(dst_hbm.at[wid], dst_v)

    def _zrow(r, carry):
        for s in range(2):
            for cc in range(IN_F // L):
                rows_v[s, r, pl.ds(cc * L, L)] = z16
        return carry
    lax.fori_loop(0, C, _zrow, 0)
    def _zw(k, carry):
        for s in range(2):
            plsc.store_scatter(w_v.at[s], [2 * k + row_base, col_lo], z16)
        return carry
    lax.fori_loop(0, C // 2, _zw, 0)

    base = s_idx * ROWS_PER_TILE
    for b in range(7):
        pltpu.sync_copy(rows_v.at[0], acc_num.at[pl.ds(base + b * C, C)])
        pltpu.sync_copy(w_v.at[0], acc_den.at[pl.ds(base + b * C, C)])
    pltpu.sync_copy(rows_v.at[0, pl.ds(0, 65)],
                    acc_num.at[pl.ds(base + 560, 65)])
    pltpu.sync_copy(w_v.at[0, pl.ds(0, 65)],
                    acc_den.at[pl.ds(base + 560, 65)])
    plsc.subcore_barrier()

    pltpu.async_copy(rows_v.at[1], acc_num.at[src_v.at[0]], sem_sr, add=True)
    pltpu.async_copy(w_v.at[1], acc_den.at[src_v.at[0]], sem_sw, add=True)
    pltpu.async_copy(wh_hbm.at[dst_v.at[0]], rows_v.at[0], sem_r)
    pltpu.async_copy(ssrc_hbm.at[src_v.at[0]], gs_v, sem_g)
    pltpu.async_copy(sdst_hbm.at[dst_v.at[0]], gd_v, sem_h)

    def _chunk(j, carry):
        p = j & 1
        pn = 1 - p
        jn = jnp.minimum(j + 1, LAST)

        pltpu.make_async_copy(ssrc_hbm.at[src_v.at[j]], gs_v, sem_g).wait()
        pltpu.make_async_copy(sdst_hbm.at[dst_v.at[j]], gd_v, sem_h).wait()

        def _wbody(k):
            ridx = 2 * k + row_base
            s1 = plsc.load_gather(gs_v, [ridx, col_lo])
            s2 = plsc.load_gather(gd_v, [ridx, col_lo])
            z = s1 + s2
            w = jnp.exp(-jnp.where(z > 0, z, ALPHA * z))
            plsc.store_scatter(w_v.at[p], [ridx, col_lo], w)
        plsc.parallel_loop(0, C // 2, unroll=4)(_wbody)

        pltpu.async_copy(ssrc_hbm.at[src_v.at[jn]], gs_v, sem_g)
        pltpu.async_copy(sdst_hbm.at[dst_v.at[jn]], gd_v, sem_h)

        pltpu.make_async_copy(wh_hbm.at[dst_v.at[j]], rows_v.at[p],
                              sem_r).wait()

        pltpu.make_async_copy(rows_v.at[pn], acc_num.at[src_v.at[j]],
                              sem_sr).wait()
        pltpu.make_async_copy(w_v.at[pn], acc_den.at[src_v.at[j]],
                              sem_sw).wait()
        pltpu.async_copy(wh_hbm.at[dst_v.at[jn]], rows_v.at[pn], sem_r)

        def _sbody(e):
            esp = iota * 0 + e
            wp = plsc.load_gather(w_v.at[p], [esp, col_lo])
            for h8 in range(HEADS):
                seg = rows_v[p, e, pl.ds(h8 * L, L)]
                rows_v[p, e, pl.ds(h8 * L, L)] = seg * wp
        plsc.parallel_loop(0, C, unroll=2)(_sbody)

        pltpu.async_copy(rows_v.at[p], acc_num.at[src_v.at[j]], sem_sr,
                         add=True)
        pltpu.async_copy(w_v.at[p], acc_den.at[src_v.at[j]], sem_sw,
                         add=True)
        return carry

    lax.fori_loop(0, NCHUNK, _chunk, 0)

    pltpu.make_async_copy(ssrc_hbm.at[src_v.at[LAST]], gs_v, sem_g).wait()
    pltpu.make_async_copy(sdst_hbm.at[dst_v.at[LAST]], gd_v, sem_h).wait()
    pltpu.make_async_copy(wh_hbm.at[dst_v.at[LAST]],
                          rows_v.at[(LAST + 1) & 1], sem_r).wait()
    pltpu.make_async_copy(rows_v.at[LAST & 1], acc_num.at[src_v.at[LAST]],
                          sem_sr).wait()
    pltpu.make_async_copy(w_v.at[LAST & 1], acc_den.at[src_v.at[LAST]],
                          sem_sw).wait()
    plsc.subcore_barrier()

    for b in range(7):
        r0 = base + b * C
        pltpu.sync_copy(acc_num.at[pl.ds(r0, C)], rows_v.at[0])
        pltpu.sync_copy(rows_v.at[0], num_hbm.at[c_idx, pl.ds(r0, C)])
        pltpu.sync_copy(acc_den.at[pl.ds(r0, C)], w_v.at[0])
        pltpu.sync_copy(w_v.at[0], den_hbm.at[c_idx, pl.ds(r0, C)])
    r0 = base + 560
    pltpu.sync_copy(acc_num.at[pl.ds(r0, 65)], rows_v.at[0, pl.ds(0, 65)])
    pltpu.sync_copy(rows_v.at[0, pl.ds(0, 65)],
                    num_hbm.at[c_idx, pl.ds(r0, 65)])
    pltpu.sync_copy(acc_den.at[pl.ds(r0, 65)], w_v.at[0, pl.ds(0, 65)])
    pltpu.sync_copy(w_v.at[0, pl.ds(0, 65)],
                    den_hbm.at[c_idx, pl.ds(r0, 65)])


def _sc_edge(wh_tab, ssrc_tab, sdst_tab, src_w, dst_w):
    mesh = plsc.VectorSubcoreMesh(core_axis_name="c", subcore_axis_name="s",
                                  num_cores=NC, num_subcores=NS)
    f = pl.kernel(
        _sc_body,
        out_type=[
            jax.ShapeDtypeStruct((NC, N, IN_F), jnp.float32),
            jax.ShapeDtypeStruct((NC, N, HEADS), jnp.float32),
        ],
        mesh=mesh,
        compiler_params=pltpu.CompilerParams(needs_layout_passes=False,
                                             use_tc_tiling_on_sc=False),
        scratch_types=[
            pltpu.VMEM((NCHUNK, C), jnp.int32),
            pltpu.VMEM((NCHUNK, C), jnp.int32),
            pltpu.VMEM((C, HEADS), jnp.float32),
            pltpu.VMEM((C, HEADS), jnp.float32),
            pltpu.VMEM((2, C, HEADS), jnp.float32),
            pltpu.VMEM((2, C, IN_F), jnp.float32),
            pltpu.VMEM_SHARED((N, IN_F), jnp.float32),
            pltpu.VMEM_SHARED((N, HEADS), jnp.float32),
            pltpu.SemaphoreType.DMA,
            pltpu.SemaphoreType.DMA,
            pltpu.SemaphoreType.DMA,
            pltpu.SemaphoreType.DMA,
            pltpu.SemaphoreType.DMA,
        ],
    )
    return f(wh_tab, ssrc_tab, sdst_tab, src_w, dst_w)


def _norm_body(num_ref, den_ref, perm_ref, out_ref):
    num = num_ref[0] + num_ref[1]
    den = den_ref[0] + den_ref[1]
    inv = 1.0 / (den + 1e-10)
    bn = num.shape[0]
    inv_hm = jnp.broadcast_to(inv[:, None, :], (bn, OUT_F, HEADS)).reshape(
        bn, IN_F)
    out_ref[...] = jnp.dot(num * inv_hm, perm_ref[...],
                           preferred_element_type=jnp.float32)


def _normalize(num, den, perm):
    BN = 1000
    return pl.pallas_call(
        _norm_body,
        grid=(N // BN,),
        in_specs=[
            pl.BlockSpec((NC, BN, IN_F), lambda i: (0, i, 0)),
            pl.BlockSpec((NC, BN, HEADS), lambda i: (0, i, 0)),
            pl.BlockSpec((IN_F, IN_F), lambda i: (0, 0)),
        ],
        out_specs=pl.BlockSpec((BN, IN_F), lambda i: (i, 0)),
        out_shape=jax.ShapeDtypeStruct((N, IN_F), jnp.float32),
    )(num, den, perm)


@jax.jit
def kernel(x, edge_index, W, a):
    w_cat = jnp.transpose(W, (1, 2, 0)).reshape(IN_F, HEADS * OUT_F)
    src = edge_index[0].reshape(NW, NCHUNK, C)
    dst = edge_index[1].reshape(NW, NCHUNK, C)

    wh_tab, ssrc_tab, sdst_tab = _dense(x, w_cat, W, a)
    num, den = _sc_edge(wh_tab, ssrc_tab, sdst_tab, src, dst)
    return _normalize(num, den, jnp.asarray(_PERM))


# speedup vs baseline: 2.0312x; 1.1243x over previous
import numpy as np

import jax
import jax.numpy as jnp
from jax import lax
from jax.experimental import pallas as pl
from jax.experimental.pallas import tpu as pltpu
from jax.experimental.pallas import tpu_sc as plsc

N = 10000
E = 320000
IN_F = 128
OUT_F = 16
HEADS = 8
ALPHA = 0.2

NC, NS, L = 2, 16, 16
NW = NC * NS
C = 80
NCHUNK = 125
EW = NCHUNK * C
ROWS_PER_TILE = N // NS
LAST = NCHUNK - 1

_PERM = np.zeros((IN_F, IN_F), np.float32)
for _k in range(OUT_F):
    for _h in range(HEADS):
        _PERM[_k * HEADS + _h, _h * OUT_F + _k] = 1.0
_EXPAND = np.zeros((HEADS, IN_F), np.float32)
for _k in range(OUT_F):
    for _h in range(HEADS):
        _EXPAND[_h, _k * HEADS + _h] = 1.0


def _dense_body(x_ref, wcat_ref, w_ref, a_ref, wh_ref, ssrc_ref, sdst_ref):
    xb = x_ref[...]
    wh_ref[...] = jnp.dot(xb, wcat_ref[...],
                          preferred_element_type=jnp.float32)
    avecs = []
    for h in range(HEADS):
        avecs.append(jnp.dot(w_ref[h], a_ref[h, :OUT_F, :],
                             preferred_element_type=jnp.float32))
    for h in range(HEADS):
        avecs.append(jnp.dot(w_ref[h], a_ref[h, OUT_F:, :],
                             preferred_element_type=jnp.float32))
    amat = jnp.concatenate(avecs, axis=1)
    ss = jnp.dot(xb, amat, preferred_element_type=jnp.float32)
    ssrc_ref[...] = ss[:, :HEADS]
    sdst_ref[...] = ss[:, HEADS:]


def _dense(x, w_cat, W, a):
    BN = 1000
    return pl.pallas_call(
        _dense_body,
        grid=(N // BN,),
        in_specs=[
            pl.BlockSpec((BN, IN_F), lambda i: (i, 0)),
            pl.BlockSpec((IN_F, IN_F), lambda i: (0, 0)),
            pl.BlockSpec((HEADS, IN_F, OUT_F), lambda i: (0, 0, 0)),
            pl.BlockSpec((HEADS, 2 * OUT_F, 1), lambda i: (0, 0, 0)),
        ],
        out_specs=[
            pl.BlockSpec((BN, IN_F), lambda i: (i, 0)),
            pl.BlockSpec((BN, HEADS), lambda i: (i, 0)),
            pl.BlockSpec((BN, HEADS), lambda i: (i, 0)),
        ],
        out_shape=[
            jax.ShapeDtypeStruct((N, IN_F), jnp.float32),
            jax.ShapeDtypeStruct((N, HEADS), jnp.float32),
            jax.ShapeDtypeStruct((N, HEADS), jnp.float32),
        ],
    )(x, w_cat, W, a)


def _sc_body(wh_hbm, ssrc_hbm, sdst_hbm, ei_hbm, num_hbm, den_hbm,
             src_v, dst_v, gs_v, gd_v, w_v, rows_v,
             acc_num, acc_den, sem_r, sem_g, sem_h, sem_sr, sem_sw):
    c_idx = lax.axis_index("c")
    s_idx = lax.axis_index("s")
    wid = s_idx * NC + c_idx

    iota = lax.iota(jnp.int32, L)
    row_base = iota >> 3
    col_lo = iota & 7
    z16 = jnp.zeros((L,), jnp.float32)

    pltpu.sync_copy(ei_hbm.at[0, pl.ds(wid * EW, EW)], src_v)
    pltpu.sync_copy(ei_hbm.at[1, pl.ds(wid * EW, EW)], dst_v)

    def _zrow(r, carry):
        for s in range(2):
            for cc in range(IN_F // L):
                rows_v[s, r, pl.ds(cc * L, L)] = z16
        return carry
    lax.fori_loop(0, C, _zrow, 0)
    def _zw(k, carry):
        for s in range(2):
            plsc.store_scatter(w_v.at[s], [2 * k + row_base, col_lo], z16)
        return carry
    lax.fori_loop(0, C // 2, _zw, 0)

    base = s_idx * ROWS_PER_TILE
    for b in range(7):
        pltpu.sync_copy(rows_v.at[0], acc_num.at[pl.ds(base + b * C, C)])
        pltpu.sync_copy(w_v.at[0], acc_den.at[pl.ds(base + b * C, C)])
    pltpu.sync_copy(rows_v.at[0, pl.ds(0, 65)],
                    acc_num.at[pl.ds(base + 560, 65)])
    pltpu.sync_copy(w_v.at[0, pl.ds(0, 65)],
                    acc_den.at[pl.ds(base + 560, 65)])
    plsc.subcore_barrier()

    pltpu.async_copy(rows_v.at[1], acc_num.at[src_v.at[pl.ds(0, C)]], sem_sr, add=True)
    pltpu.async_copy(w_v.at[1], acc_den.at[src_v.at[pl.ds(0, C)]], sem_sw, add=True)
    pltpu.async_copy(wh_hbm.at[dst_v.at[pl.ds(0, C)]], rows_v.at[0], sem_r)
    pltpu.async_copy(ssrc_hbm.at[src_v.at[pl.ds(0, C)]], gs_v, sem_g)
    pltpu.async_copy(sdst_hbm.at[dst_v.at[pl.ds(0, C)]], gd_v, sem_h)

    def _chunk(j, carry):
        p = j & 1
        pn = 1 - p
        jn = jnp.minimum(j + 1, LAST)

        pltpu.make_async_copy(ssrc_hbm.at[src_v.at[pl.ds(j * C, C)]], gs_v, sem_g).wait()
        pltpu.make_async_copy(sdst_hbm.at[dst_v.at[pl.ds(j * C, C)]], gd_v, sem_h).wait()

        def _wbody(k):
            ridx = 2 * k + row_base
            s1 = plsc.load_gather(gs_v, [ridx, col_lo])
            s2 = plsc.load_gather(gd_v, [ridx, col_lo])
            z = s1 + s2
            w = jnp.exp(-jnp.where(z > 0, z, ALPHA * z))
            plsc.store_scatter(w_v.at[p], [ridx, col_lo], w)
        plsc.parallel_loop(0, C // 2, unroll=4)(_wbody)

        pltpu.async_copy(ssrc_hbm.at[src_v.at[pl.ds(jn * C, C)]], gs_v, sem_g)
        pltpu.async_copy(sdst_hbm.at[dst_v.at[pl.ds(jn * C, C)]], gd_v, sem_h)

        pltpu.make_async_copy(wh_hbm.at[dst_v.at[pl.ds(j * C, C)]], rows_v.at[p],
                              sem_r).wait()

        pltpu.make_async_copy(rows_v.at[pn], acc_num.at[src_v.at[pl.ds(j * C, C)]],
                              sem_sr).wait()
        pltpu.make_async_copy(w_v.at[pn], acc_den.at[src_v.at[pl.ds(j * C, C)]],
                              sem_sw).wait()
        pltpu.async_copy(wh_hbm.at[dst_v.at[pl.ds(jn * C, C)]], rows_v.at[pn], sem_r)

        def _sbody(e):
            esp = iota * 0 + e
            wp = plsc.load_gather(w_v.at[p], [esp, col_lo])
            for h8 in range(HEADS):
                seg = rows_v[p, e, pl.ds(h8 * L, L)]
                rows_v[p, e, pl.ds(h8 * L, L)] = seg * wp
        plsc.parallel_loop(0, C, unroll=2)(_sbody)

        pltpu.async_copy(rows_v.at[p], acc_num.at[src_v.at[pl.ds(j * C, C)]], sem_sr,
                         add=True)
        pltpu.async_copy(w_v.at[p], acc_den.at[src_v.at[pl.ds(j * C, C)]], sem_sw,
                         add=True)
        return carry

    lax.fori_loop(0, NCHUNK, _chunk, 0)

    pltpu.make_async_copy(ssrc_hbm.at[src_v.at[pl.ds(LAST * C, C)]], gs_v, sem_g).wait()
    pltpu.make_async_copy(sdst_hbm.at[dst_v.at[pl.ds(LAST * C, C)]], gd_v, sem_h).wait()
    pltpu.make_async_copy(wh_hbm.at[dst_v.at[pl.ds(LAST * C, C)]],
                          rows_v.at[(LAST + 1) & 1], sem_r).wait()
    pltpu.make_async_copy(rows_v.at[LAST & 1], acc_num.at[src_v.at[pl.ds(LAST * C, C)]],
                          sem_sr).wait()
    pltpu.make_async_copy(w_v.at[LAST & 1], acc_den.at[src_v.at[pl.ds(LAST * C, C)]],
                          sem_sw).wait()
    plsc.subcore_barrier()

    for b in range(7):
        r0 = base + b * C
        pltpu.sync_copy(acc_num.at[pl.ds(r0, C)], rows_v.at[0])
        pltpu.sync_copy(rows_v.at[0], num_hbm.at[c_idx, pl.ds(r0, C)])
        pltpu.sync_copy(acc_den.at[pl.ds(r0, C)], w_v.at[0])
        pltpu.sync_copy(w_v.at[0], den_hbm.at[c_idx, pl.ds(r0, C)])
    r0 = base + 560
    pltpu.sync_copy(acc_num.at[pl.ds(r0, 65)], rows_v.at[0, pl.ds(0, 65)])
    pltpu.sync_copy(rows_v.at[0, pl.ds(0, 65)],
                    num_hbm.at[c_idx, pl.ds(r0, 65)])
    pltpu.sync_copy(acc_den.at[pl.ds(r0, 65)], w_v.at[0, pl.ds(0, 65)])
    pltpu.sync_copy(w_v.at[0, pl.ds(0, 65)],
                    den_hbm.at[c_idx, pl.ds(r0, 65)])


def _sc_edge(wh_tab, ssrc_tab, sdst_tab, edge_index):
    mesh = plsc.VectorSubcoreMesh(core_axis_name="c", subcore_axis_name="s",
                                  num_cores=NC, num_subcores=NS)
    f = pl.kernel(
        _sc_body,
        out_type=[
            jax.ShapeDtypeStruct((NC, N, IN_F), jnp.float32),
            jax.ShapeDtypeStruct((NC, N, HEADS), jnp.float32),
        ],
        mesh=mesh,
        compiler_params=pltpu.CompilerParams(needs_layout_passes=False,
                                             use_tc_tiling_on_sc=False),
        scratch_types=[
            pltpu.VMEM((EW,), jnp.int32),
            pltpu.VMEM((EW,), jnp.int32),
            pltpu.VMEM((C, HEADS), jnp.float32),
            pltpu.VMEM((C, HEADS), jnp.float32),
            pltpu.VMEM((2, C, HEADS), jnp.float32),
            pltpu.VMEM((2, C, IN_F), jnp.float32),
            pltpu.VMEM_SHARED((N, IN_F), jnp.float32),
            pltpu.VMEM_SHARED((N, HEADS), jnp.float32),
            pltpu.SemaphoreType.DMA,
            pltpu.SemaphoreType.DMA,
            pltpu.SemaphoreType.DMA,
            pltpu.SemaphoreType.DMA,
            pltpu.SemaphoreType.DMA,
        ],
    )
    return f(wh_tab, ssrc_tab, sdst_tab, edge_index)


def _norm_body(num_ref, den_ref, perm_ref, exp_ref, out_ref):
    num = num_ref[0] + num_ref[1]
    den = den_ref[0] + den_ref[1]
    inv = 1.0 / (den + 1e-10)
    inv_hm = jnp.dot(inv, exp_ref[...], preferred_element_type=jnp.float32)
    out_ref[...] = jnp.dot(num * inv_hm, perm_ref[...],
                           preferred_element_type=jnp.float32)


def _normalize(num, den, perm, expand):
    BN = 1000
    return pl.pallas_call(
        _norm_body,
        grid=(N // BN,),
        in_specs=[
            pl.BlockSpec((NC, BN, IN_F), lambda i: (0, i, 0)),
            pl.BlockSpec((NC, BN, HEADS), lambda i: (0, i, 0)),
            pl.BlockSpec((IN_F, IN_F), lambda i: (0, 0)),
            pl.BlockSpec((HEADS, IN_F), lambda i: (0, 0)),
        ],
        out_specs=pl.BlockSpec((BN, IN_F), lambda i: (i, 0)),
        out_shape=jax.ShapeDtypeStruct((N, IN_F), jnp.float32),
    )(num, den, perm, expand)


@jax.jit
def kernel(x, edge_index, W, a):
    w_cat = jnp.transpose(W, (1, 2, 0)).reshape(IN_F, HEADS * OUT_F)

    wh_tab, ssrc_tab, sdst_tab = _dense(x, w_cat, W, a)
    num, den = _sc_edge(wh_tab, ssrc_tab, sdst_tab, edge_index)
    return _normalize(num, den, jnp.asarray(_PERM), jnp.asarray(_EXPAND))
